# baseline (speedup 1.0000x reference)
import functools

import jax
import jax.numpy as jnp
from jax import lax
from jax.experimental import pallas as pl
from jax.experimental.pallas import tpu as pltpu
from jax.experimental.pallas import tpu_sc as plsc

N_NODES = 1024
N_EDGES = 65536
IN_DIM = 4
HID = 8
LANES = 16
N_SUB = 16
NODES_PER_TILE = N_NODES // N_SUB
CHUNK = 128
ROWS_PER_TILE = (N_EDGES // CHUNK) // N_SUB
EDGES_PER_TILE = N_EDGES // N_SUB


def _rsqrt16(d):
    i = plsc.bitcast(d, jnp.int32)
    i = 0x5F3759DF - lax.shift_right_logical(i, 1)
    y = plsc.bitcast(i, jnp.float32)
    for _ in range(3):
        y = y * (1.5 - 0.5 * d * y * y)
    return y


NBUF = 8
N_ROUNDS = ROWS_PER_TILE // NBUF


def _gcn_body(src_hbm, dst_hbm, x_hbm, w1_hbm, b1_hbm, w2_hbm, b2_hbm,
              ones_hbm, out_hbm,
              srcl, dstl, rowsb, onesv, xloc, w1v, b1v, w2v, b2v,
              degb, dinvb, xwb, hwb, accb, pubb, zb, outb,
              t_src, t_acc, sem_g, sem_s):
    c = lax.axis_index("c")
    s = lax.axis_index("s")
    nd = pl.ds(s * NODES_PER_TILE, NODES_PER_TILE)

    scope_stage = jax.named_scope("stage")
    scope_stage.__enter__()
    stage = [
        pltpu.async_copy(src_hbm.at[pl.ds(s * EDGES_PER_TILE, EDGES_PER_TILE)],
                         srcl, sem_g),
        pltpu.async_copy(dst_hbm.at[pl.ds(s * EDGES_PER_TILE, EDGES_PER_TILE)],
                         dstl, sem_g),
        pltpu.async_copy(ones_hbm, onesv, sem_g),
        pltpu.async_copy(x_hbm.at[nd], xloc, sem_g),
        pltpu.async_copy(w1_hbm, w1v, sem_g),
        pltpu.async_copy(b1_hbm, b1v, sem_g),
        pltpu.async_copy(w2_hbm, w2v, sem_g),
        pltpu.async_copy(b2_hbm, b2v, sem_g),
    ]
    for d in stage:
        d.wait()
    pltpu.sync_copy(onesv.at[pl.ds(0, NODES_PER_TILE)], t_acc.at[nd])
    pltpu.sync_copy(onesv.at[pl.ds(0, NODES_PER_TILE)], t_src.at[nd])
    plsc.subcore_barrier()
    scope_stage.__exit__(None, None, None)

    scope_deg = jax.named_scope("deg_mac")
    scope_deg.__enter__()
    deg_g = pltpu.async_copy(t_src.at[srcl], rowsb, sem_g)

    def mac1(i, carry):
        xrow = xloc[i]
        acc = jnp.zeros((LANES,), jnp.float32)
        for k in range(IN_DIM):
            acc = acc + w1v[k] * xrow[k]
        xwb[i] = acc
        zb[i] = jnp.zeros((LANES,), jnp.float32)
        return carry
    lax.fori_loop(0, NODES_PER_TILE, mac1, 0)
    deg_g.wait()
    pltpu.async_copy(rowsb, t_acc.at[dstl], sem_s, add=True).wait()
    plsc.subcore_barrier()
    scope_deg.__exit__(None, None, None)

    scope_pub = jax.named_scope("dinv_pub")
    scope_pub.__enter__()
    pltpu.sync_copy(t_acc.at[nd], degb)
    b1 = b1v[...]
    b2 = b2v[...]

    def loc1(i, carry):
        y = _rsqrt16(degb[i])
        dinvb[i] = y
        pubb[i] = xwb[i] * y
        return carry
    lax.fori_loop(0, NODES_PER_TILE, loc1, 0)
    pltpu.sync_copy(pubb, t_src.at[nd])
    pltpu.sync_copy(zb, t_acc.at[nd])
    plsc.subcore_barrier()
    scope_pub.__exit__(None, None, None)

    def edge_pass():
        pltpu.async_copy(t_src.at[srcl], rowsb, sem_g).wait()
        pltpu.async_copy(rowsb, t_acc.at[dstl], sem_s, add=True).wait()

    with jax.named_scope("l1_edges"):
        edge_pass()
    plsc.subcore_barrier()

    scope_mid = jax.named_scope("mid_locals")
    scope_mid.__enter__()
    pltpu.sync_copy(t_acc.at[nd], accb)

    def loc2(i, carry):
        y = dinvb[i]
        h1 = jnp.maximum(y * accb[i] + (y * y) * xwb[i] + b1, 0.0)
        accb[i] = h1
        return carry
    lax.fori_loop(0, NODES_PER_TILE, loc2, 0)

    def loc3(i, carry):
        h1row = accb[i]
        acc = jnp.zeros((LANES,), jnp.float32)
        for k in range(HID):
            acc = acc + w2v[k] * h1row[k]
        hwb[i] = acc
        pubb[i] = acc * dinvb[i]
        return carry
    lax.fori_loop(0, NODES_PER_TILE, loc3, 0)
    pltpu.sync_copy(pubb, t_src.at[nd])
    pltpu.sync_copy(zb, t_acc.at[nd])
    plsc.subcore_barrier()
    scope_mid.__exit__(None, None, None)

    with jax.named_scope("l2_edges"):
        edge_pass()
    plsc.subcore_barrier()

    scope_fin = jax.named_scope("final")
    scope_fin.__enter__()
    pltpu.sync_copy(t_acc.at[nd], accb)

    def loc4(i, carry):
        y = dinvb[i]
        outb[i] = y * accb[i] + (y * y) * hwb[i] + b2
        return carry
    lax.fori_loop(0, NODES_PER_TILE, loc4, 0)

    @pl.when(c == 0)
    def _():
        pltpu.sync_copy(outb, out_hbm.at[nd])
    scope_fin.__exit__(None, None, None)


def _gcn_sc(src2, dst2, x, w1p, b1p, w2p, b2p, ones):
    mesh = plsc.VectorSubcoreMesh(core_axis_name="c", subcore_axis_name="s",
                                  num_cores=1)
    f32 = jnp.float32
    kern = pl.kernel(
        _gcn_body,
        out_type=jax.ShapeDtypeStruct((N_NODES, LANES), f32),
        mesh=mesh,
        compiler_params=pltpu.CompilerParams(needs_layout_passes=False,
                                             use_tc_tiling_on_sc=False),
        scratch_types=[
            pltpu.VMEM((EDGES_PER_TILE,), jnp.int32),
            pltpu.VMEM((EDGES_PER_TILE,), jnp.int32),
            pltpu.VMEM((EDGES_PER_TILE, LANES), f32),
            pltpu.VMEM((CHUNK, LANES), f32),
            pltpu.VMEM((NODES_PER_TILE, LANES), f32),
            pltpu.VMEM((IN_DIM, LANES), f32),
            pltpu.VMEM((LANES,), f32),
            pltpu.VMEM((HID, LANES), f32),
            pltpu.VMEM((LANES,), f32),
            pltpu.VMEM((NODES_PER_TILE, LANES), f32),
            pltpu.VMEM((NODES_PER_TILE, LANES), f32),
            pltpu.VMEM((NODES_PER_TILE, LANES), f32),
            pltpu.VMEM((NODES_PER_TILE, LANES), f32),
            pltpu.VMEM((NODES_PER_TILE, LANES), f32),
            pltpu.VMEM((NODES_PER_TILE, LANES), f32),
            pltpu.VMEM((NODES_PER_TILE, LANES), f32),
            pltpu.VMEM((NODES_PER_TILE, LANES), f32),
            pltpu.VMEM_SHARED((N_NODES, LANES), f32),
            pltpu.VMEM_SHARED((N_NODES, LANES), f32),
            pltpu.SemaphoreType.DMA,
            pltpu.SemaphoreType.DMA,
        ],
    )
    return kern(src2, dst2, x, w1p, b1p, w2p, b2p, ones)


KBLK = 512
NSLOT = 4


def _mlp_body(v_ref, w1_hbm, b1_ref, w3_ref, b3_ref, o_ref, wbuf, sems):
    n_in = w1_hbm.shape[0]
    n_hidden = w1_hbm.shape[1]
    n_slab = n_in // KBLK

    def fire(i):
        return pltpu.async_copy(
            w1_hbm.at[pl.ds(i * KBLK, KBLK), :], wbuf.at[i % NSLOT],
            sems.at[i % NSLOT])

    descs = [fire(i) for i in range(NSLOT)]
    u = jnp.zeros((1, n_hidden), jnp.float32)
    for i in range(n_slab):
        descs[i % NSLOT].wait()
        vblk = v_ref[:, pl.ds(i * KBLK, KBLK)]
        u = u + jnp.dot(vblk, wbuf[i % NSLOT],
                        preferred_element_type=jnp.float32)
        if i + NSLOT < n_slab:
            descs[i % NSLOT] = fire(i + NSLOT)
    u = jnp.maximum(u + b1_ref[...], 0.0)
    z = jnp.dot(u, w3_ref[...], preferred_element_type=jnp.float32)
    z = z + b3_ref[...]
    z = z - jnp.max(z, axis=-1, keepdims=True)
    e = jnp.exp(z)
    o_ref[...] = e / jnp.sum(e, axis=-1, keepdims=True)


def _mlp_tc(v, lin1_W, lin1_b, lin3_W, lin3_b):
    n_hidden = lin1_W.shape[1]
    n_out = lin3_W.shape[1]
    return pl.pallas_call(
        _mlp_body,
        in_specs=[
            pl.BlockSpec(memory_space=pltpu.VMEM),
            pl.BlockSpec(memory_space=pl.ANY),
            pl.BlockSpec(memory_space=pltpu.VMEM),
            pl.BlockSpec(memory_space=pltpu.VMEM),
            pl.BlockSpec(memory_space=pltpu.VMEM),
        ],
        out_specs=pl.BlockSpec(memory_space=pltpu.VMEM),
        out_shape=jax.ShapeDtypeStruct((1, n_out), jnp.float32),
        scratch_shapes=[
            pltpu.VMEM((NSLOT, KBLK, n_hidden), jnp.float32),
            pltpu.SemaphoreType.DMA((NSLOT,)),
        ],
    )(v, lin1_W, lin1_b, lin3_W, lin3_b)


def kernel(x, edge_index, W1, b1, W2, b2, lin1_W, lin1_b, lin3_W, lin3_b):
    f32 = jnp.float32
    src2 = edge_index[0]
    dst2 = edge_index[1]
    x16 = jnp.zeros((N_NODES, LANES), f32).at[:, :IN_DIM].set(x)
    w1p = jnp.zeros((IN_DIM, LANES), f32).at[:, :HID].set(W1)
    b1p = jnp.zeros((LANES,), f32).at[:HID].set(b1)
    w2p = jnp.zeros((HID, LANES), f32).at[:, :HID].set(W2)
    b2p = jnp.zeros((LANES,), f32).at[:HID].set(b2)
    ones = jnp.ones((CHUNK, LANES), f32)
    h2 = _gcn_sc(src2, dst2, x16, w1p, b1p, w2p, b2p, ones)
    v = h2[:, :HID].reshape(1, -1)
    out = _mlp_tc(v, lin1_W, lin1_b.reshape(1, -1), lin3_W, lin3_b.reshape(1, -1))
    return out.reshape(-1)


# speedup vs baseline: 1.0203x; 1.0203x over previous
import functools

import jax
import jax.numpy as jnp
from jax import lax
from jax.experimental import pallas as pl
from jax.experimental.pallas import tpu as pltpu
from jax.experimental.pallas import tpu_sc as plsc

N_NODES = 1024
N_EDGES = 65536
IN_DIM = 4
HID = 8
LANES = 16
N_SUB = 16
NODES_PER_TILE = N_NODES // N_SUB
CHUNK = 128
ROWS_PER_TILE = (N_EDGES // CHUNK) // N_SUB
EDGES_PER_TILE = N_EDGES // N_SUB


def _rsqrt16(d):
    i = plsc.bitcast(d, jnp.int32)
    i = 0x5F3759DF - lax.shift_right_logical(i, 1)
    y = plsc.bitcast(i, jnp.float32)
    for _ in range(3):
        y = y * (1.5 - 0.5 * d * y * y)
    return y


NBUF = 8
N_ROUNDS = ROWS_PER_TILE // NBUF


def _gcn_body(src_hbm, dst_hbm, x_hbm, w1_hbm, b1_hbm, w2_hbm, b2_hbm,
              out_hbm,
              srcl, dstl, rowsb, onesv, xloc, w1v, b1v, w2v, b2v,
              degb, dinvb, xwb, hwb, accb, pubb, zb, outb,
              t_src, t_acc, sem_g, sem_s):
    c = lax.axis_index("c")
    s = lax.axis_index("s")
    nd = pl.ds(s * NODES_PER_TILE, NODES_PER_TILE)

    scope_stage = jax.named_scope("stage")
    scope_stage.__enter__()
    stage = [
        pltpu.async_copy(src_hbm.at[pl.ds(s * EDGES_PER_TILE, EDGES_PER_TILE)],
                         srcl, sem_g),
        pltpu.async_copy(dst_hbm.at[pl.ds(s * EDGES_PER_TILE, EDGES_PER_TILE)],
                         dstl, sem_g),
        pltpu.async_copy(x_hbm.at[nd], xloc, sem_g),
        pltpu.async_copy(w1_hbm, w1v, sem_g),
        pltpu.async_copy(b1_hbm, b1v, sem_g),
        pltpu.async_copy(w2_hbm, w2v, sem_g),
        pltpu.async_copy(b2_hbm, b2v, sem_g),
    ]
    def fill_ones(i, carry):
        onesv[i] = jnp.full((LANES,), 1.0, jnp.float32)
        return carry
    lax.fori_loop(0, NODES_PER_TILE, fill_ones, 0)
    for d in stage:
        d.wait()
    pltpu.sync_copy(onesv, t_acc.at[nd])
    pltpu.sync_copy(onesv, t_src.at[nd])
    plsc.subcore_barrier()
    scope_stage.__exit__(None, None, None)

    scope_deg = jax.named_scope("deg_mac")
    scope_deg.__enter__()
    deg_g = pltpu.async_copy(t_src.at[srcl], rowsb, sem_g)

    def mac1(i, carry):
        xrow = xloc[i]
        acc = jnp.zeros((LANES,), jnp.float32)
        for k in range(IN_DIM):
            acc = acc + w1v[k] * xrow[k]
        xwb[i] = acc
        zb[i] = jnp.zeros((LANES,), jnp.float32)
        return carry
    lax.fori_loop(0, NODES_PER_TILE, mac1, 0)
    deg_g.wait()
    pltpu.async_copy(rowsb, t_acc.at[dstl], sem_s, add=True).wait()
    plsc.subcore_barrier()
    scope_deg.__exit__(None, None, None)

    scope_pub = jax.named_scope("dinv_pub")
    scope_pub.__enter__()
    pltpu.sync_copy(t_acc.at[nd], degb)
    b1 = b1v[...]
    b2 = b2v[...]

    def loc1(i, carry):
        y = _rsqrt16(degb[i])
        dinvb[i] = y
        pubb[i] = xwb[i] * y
        return carry
    lax.fori_loop(0, NODES_PER_TILE, loc1, 0)
    pltpu.sync_copy(pubb, t_src.at[nd])
    pltpu.sync_copy(zb, t_acc.at[nd])
    plsc.subcore_barrier()
    scope_pub.__exit__(None, None, None)

    HALF = EDGES_PER_TILE // 2

    def edge_pass():
        ga = pltpu.async_copy(t_src.at[srcl.at[pl.ds(0, HALF)]],
                              rowsb.at[pl.ds(0, HALF)], sem_g)
        ga.wait()
        sa = pltpu.async_copy(rowsb.at[pl.ds(0, HALF)],
                              t_acc.at[dstl.at[pl.ds(0, HALF)]],
                              sem_s, add=True)
        gb = pltpu.async_copy(t_src.at[srcl.at[pl.ds(HALF, HALF)]],
                              rowsb.at[pl.ds(HALF, HALF)], sem_g)
        gb.wait()
        sa.wait()
        pltpu.async_copy(rowsb.at[pl.ds(HALF, HALF)],
                         t_acc.at[dstl.at[pl.ds(HALF, HALF)]],
                         sem_s, add=True).wait()

    with jax.named_scope("l1_edges"):
        edge_pass()
    plsc.subcore_barrier()

    scope_mid = jax.named_scope("mid_locals")
    scope_mid.__enter__()
    pltpu.sync_copy(t_acc.at[nd], accb)

    def loc2(i, carry):
        y = dinvb[i]
        h1 = jnp.maximum(y * accb[i] + (y * y) * xwb[i] + b1, 0.0)
        acc = jnp.zeros((LANES,), jnp.float32)
        for k in range(HID):
            acc = acc + w2v[k] * h1[k]
        hwb[i] = acc
        pubb[i] = acc * y
        return carry
    lax.fori_loop(0, NODES_PER_TILE, loc2, 0)
    pltpu.sync_copy(pubb, t_src.at[nd])
    pltpu.sync_copy(zb, t_acc.at[nd])
    plsc.subcore_barrier()
    scope_mid.__exit__(None, None, None)

    with jax.named_scope("l2_edges"):
        edge_pass()
    plsc.subcore_barrier()

    scope_fin = jax.named_scope("final")
    scope_fin.__enter__()
    pltpu.sync_copy(t_acc.at[nd], accb)

    lanes = lax.iota(jnp.int32, LANES)
    lo = lanes < HID

    def loc4(i, carry):
        y = dinvb[i]
        o = y * accb[i] + (y * y) * hwb[i] + b2
        plsc.store_scatter(outb, [i * HID + lanes], o, mask=lo)
        return carry
    lax.fori_loop(0, NODES_PER_TILE, loc4, 0)

    @pl.when(c == 0)
    def _():
        pltpu.sync_copy(outb,
                        out_hbm.at[pl.ds(s * NODES_PER_TILE * HID,
                                         NODES_PER_TILE * HID)])
    scope_fin.__exit__(None, None, None)


def _gcn_sc(src2, dst2, x, w1p, b1p, w2p, b2p):
    mesh = plsc.VectorSubcoreMesh(core_axis_name="c", subcore_axis_name="s",
                                  num_cores=1)
    f32 = jnp.float32
    kern = pl.kernel(
        _gcn_body,
        out_type=jax.ShapeDtypeStruct((N_NODES * HID,), f32),
        mesh=mesh,
        compiler_params=pltpu.CompilerParams(needs_layout_passes=False,
                                             use_tc_tiling_on_sc=False),
        scratch_types=[
            pltpu.VMEM((EDGES_PER_TILE,), jnp.int32),
            pltpu.VMEM((EDGES_PER_TILE,), jnp.int32),
            pltpu.VMEM((EDGES_PER_TILE, LANES), f32),
            pltpu.VMEM((NODES_PER_TILE, LANES), f32),
            pltpu.VMEM((NODES_PER_TILE, LANES), f32),
            pltpu.VMEM((IN_DIM, LANES), f32),
            pltpu.VMEM((LANES,), f32),
            pltpu.VMEM((HID, LANES), f32),
            pltpu.VMEM((LANES,), f32),
            pltpu.VMEM((NODES_PER_TILE, LANES), f32),
            pltpu.VMEM((NODES_PER_TILE, LANES), f32),
            pltpu.VMEM((NODES_PER_TILE, LANES), f32),
            pltpu.VMEM((NODES_PER_TILE, LANES), f32),
            pltpu.VMEM((NODES_PER_TILE, LANES), f32),
            pltpu.VMEM((NODES_PER_TILE, LANES), f32),
            pltpu.VMEM((NODES_PER_TILE, LANES), f32),
            pltpu.VMEM((NODES_PER_TILE * HID,), f32),
            pltpu.VMEM_SHARED((N_NODES, LANES), f32),
            pltpu.VMEM_SHARED((N_NODES, LANES), f32),
            pltpu.SemaphoreType.DMA,
            pltpu.SemaphoreType.DMA,
        ],
    )
    return kern(src2, dst2, x, w1p, b1p, w2p, b2p)


KBLK = 512
NSLOT = 4


def _mlp_body(v_ref, w1_hbm, b1_ref, w3_ref, b3_ref, o_ref, wbuf, sems):
    n_in = w1_hbm.shape[0]
    n_hidden = w1_hbm.shape[1]
    n_slab = n_in // KBLK

    def fire(i):
        return pltpu.async_copy(
            w1_hbm.at[pl.ds(i * KBLK, KBLK), :], wbuf.at[i % NSLOT],
            sems.at[i % NSLOT])

    descs = [fire(i) for i in range(NSLOT)]
    u = jnp.zeros((1, n_hidden), jnp.float32)
    for i in range(n_slab):
        descs[i % NSLOT].wait()
        vblk = v_ref[:, pl.ds(i * KBLK, KBLK)]
        u = u + jnp.dot(vblk, wbuf[i % NSLOT],
                        preferred_element_type=jnp.float32)
        if i + NSLOT < n_slab:
            descs[i % NSLOT] = fire(i + NSLOT)
    u = jnp.maximum(u + b1_ref[...], 0.0)
    z = jnp.dot(u, w3_ref[...], preferred_element_type=jnp.float32)
    z = z + b3_ref[...]
    z = z - jnp.max(z, axis=-1, keepdims=True)
    e = jnp.exp(z)
    o_ref[...] = e / jnp.sum(e, axis=-1, keepdims=True)


def _mlp_tc(v, lin1_W, lin1_b, lin3_W, lin3_b):
    n_hidden = lin1_W.shape[1]
    n_out = lin3_W.shape[1]
    return pl.pallas_call(
        _mlp_body,
        in_specs=[
            pl.BlockSpec(memory_space=pltpu.VMEM),
            pl.BlockSpec(memory_space=pl.ANY),
            pl.BlockSpec(memory_space=pltpu.VMEM),
            pl.BlockSpec(memory_space=pltpu.VMEM),
            pl.BlockSpec(memory_space=pltpu.VMEM),
        ],
        out_specs=pl.BlockSpec(memory_space=pltpu.VMEM),
        out_shape=jax.ShapeDtypeStruct((1, n_out), jnp.float32),
        scratch_shapes=[
            pltpu.VMEM((NSLOT, KBLK, n_hidden), jnp.float32),
            pltpu.SemaphoreType.DMA((NSLOT,)),
        ],
    )(v, lin1_W, lin1_b, lin3_W, lin3_b)


def kernel(x, edge_index, W1, b1, W2, b2, lin1_W, lin1_b, lin3_W, lin3_b):
    f32 = jnp.float32
    src2 = edge_index[0]
    dst2 = edge_index[1]
    x16 = jnp.zeros((N_NODES, LANES), f32).at[:, :IN_DIM].set(x)
    w1p = jnp.zeros((IN_DIM, LANES), f32).at[:, :HID].set(W1)
    b1p = jnp.zeros((LANES,), f32).at[:HID].set(b1)
    w2p = jnp.zeros((HID, LANES), f32).at[:, :HID].set(W2)
    b2p = jnp.zeros((LANES,), f32).at[:HID].set(b2)
    h2 = _gcn_sc(src2, dst2, x16, w1p, b1p, w2p, b2p)
    v = h2.reshape(1, -1)
    out = _mlp_tc(v, lin1_W, lin1_b.reshape(1, -1), lin3_W, lin3_b.reshape(1, -1))
    return out.reshape(-1)


# speedup vs baseline: 1.0944x; 1.0725x over previous
import functools

import jax
import jax.numpy as jnp
from jax import lax
from jax.experimental import pallas as pl
from jax.experimental.pallas import tpu as pltpu
from jax.experimental.pallas import tpu_sc as plsc

N_NODES = 1024
N_EDGES = 65536
IN_DIM = 4
HID = 8
LANES = 16
N_SUB = 16
NODES_PER_TILE = N_NODES // N_SUB
CHUNK = 128
ROWS_PER_TILE = (N_EDGES // CHUNK) // N_SUB
EDGES_PER_TILE = N_EDGES // N_SUB


def _rsqrt16(d):
    i = plsc.bitcast(d, jnp.int32)
    i = 0x5F3759DF - lax.shift_right_logical(i, 1)
    y = plsc.bitcast(i, jnp.float32)
    for _ in range(3):
        y = y * (1.5 - 0.5 * d * y * y)
    return y


NBUF = 8
N_ROUNDS = ROWS_PER_TILE // NBUF


def _gcn_body(edge_hbm, x_hbm, params_hbm,
              out_hbm,
              srcl, dstl, rowsb, onesv, xloc, pv,
              degb, dinvb, xwb, hwb, accb, pubb, zb, outb,
              t_src, t_acc, sem_g, sem_s):
    c = lax.axis_index("c")
    s = lax.axis_index("s")
    nd = pl.ds(s * NODES_PER_TILE, NODES_PER_TILE)

    scope_stage = jax.named_scope("stage")
    scope_stage.__enter__()
    stage = [
        pltpu.async_copy(
            edge_hbm.at[0, pl.ds(s * EDGES_PER_TILE, EDGES_PER_TILE)],
            srcl, sem_g),
        pltpu.async_copy(
            edge_hbm.at[1, pl.ds(s * EDGES_PER_TILE, EDGES_PER_TILE)],
            dstl, sem_g),
        pltpu.async_copy(x_hbm.at[s], xloc, sem_g),
        pltpu.async_copy(params_hbm, pv, sem_g),
    ]
    def fill_ones(i, carry):
        onesv[i] = jnp.full((LANES,), 1.0, jnp.float32)
        return carry
    lax.fori_loop(0, NODES_PER_TILE, fill_ones, 0)
    for d in stage:
        d.wait()
    pltpu.sync_copy(onesv, t_acc.at[nd])
    pltpu.sync_copy(onesv, t_src.at[nd])
    plsc.subcore_barrier()
    scope_stage.__exit__(None, None, None)

    scope_deg = jax.named_scope("deg_mac")
    scope_deg.__enter__()
    deg_g = pltpu.async_copy(t_src.at[srcl], rowsb, sem_g)

    w1r = [pv[k] for k in range(IN_DIM)]
    zero16 = jnp.zeros((LANES,), jnp.float32)

    def mac1(j, carry):
        xv = xloc[pl.ds(j * LANES, LANES)]
        for m in range(4):
            acc = zero16
            for k in range(IN_DIM):
                acc = acc + w1r[k] * xv[4 * m + k]
            xwb[4 * j + m] = acc
            zb[4 * j + m] = zero16
        return carry
    lax.fori_loop(0, NODES_PER_TILE // 4, mac1, 0)
    deg_g.wait()
    pltpu.async_copy(rowsb, t_acc.at[dstl], sem_s, add=True).wait()
    plsc.subcore_barrier()
    scope_deg.__exit__(None, None, None)

    scope_pub = jax.named_scope("dinv_pub")
    scope_pub.__enter__()
    pltpu.sync_copy(t_acc.at[nd], degb)
    b1 = pv[4]
    b2 = pv[13]
    w2r = [pv[5 + k] for k in range(HID)]

    def loc1(i, carry):
        y = _rsqrt16(degb[i])
        dinvb[i] = y
        pubb[i] = xwb[i] * y
        return carry
    lax.fori_loop(0, NODES_PER_TILE, loc1, 0)
    pltpu.sync_copy(pubb, t_src.at[nd])
    pltpu.sync_copy(zb, t_acc.at[nd])
    plsc.subcore_barrier()
    scope_pub.__exit__(None, None, None)

    HALF = EDGES_PER_TILE // 2

    def edge_pass():
        ga = pltpu.async_copy(t_src.at[srcl.at[pl.ds(0, HALF)]],
                              rowsb.at[pl.ds(0, HALF)], sem_g)
        ga.wait()
        sa = pltpu.async_copy(rowsb.at[pl.ds(0, HALF)],
                              t_acc.at[dstl.at[pl.ds(0, HALF)]],
                              sem_s, add=True)
        gb = pltpu.async_copy(t_src.at[srcl.at[pl.ds(HALF, HALF)]],
                              rowsb.at[pl.ds(HALF, HALF)], sem_g)
        gb.wait()
        sa.wait()
        pltpu.async_copy(rowsb.at[pl.ds(HALF, HALF)],
                         t_acc.at[dstl.at[pl.ds(HALF, HALF)]],
                         sem_s, add=True).wait()

    with jax.named_scope("l1_edges"):
        edge_pass()
    plsc.subcore_barrier()

    scope_mid = jax.named_scope("mid_locals")
    scope_mid.__enter__()
    pltpu.sync_copy(t_acc.at[nd], accb)

    def loc2(i, carry):
        y = dinvb[i]
        h1 = jnp.maximum(y * accb[i] + (y * y) * xwb[i] + b1, 0.0)
        acc = jnp.zeros((LANES,), jnp.float32)
        for k in range(HID):
            acc = acc + w2r[k] * h1[k]
        hwb[i] = acc
        pubb[i] = acc * y
        return carry
    lax.fori_loop(0, NODES_PER_TILE, loc2, 0)
    pltpu.sync_copy(pubb, t_src.at[nd])
    pltpu.sync_copy(zb, t_acc.at[nd])
    plsc.subcore_barrier()
    scope_mid.__exit__(None, None, None)

    with jax.named_scope("l2_edges"):
        edge_pass()
    plsc.subcore_barrier()

    scope_fin = jax.named_scope("final")
    scope_fin.__enter__()
    pltpu.sync_copy(t_acc.at[nd], accb)

    lanes = lax.iota(jnp.int32, LANES)
    lo = lanes < HID

    def loc4(i, carry):
        y = dinvb[i]
        o = y * accb[i] + (y * y) * hwb[i] + b2
        plsc.store_scatter(outb, [i * HID + lanes], o, mask=lo)
        return carry
    lax.fori_loop(0, NODES_PER_TILE, loc4, 0)

    @pl.when(c == 0)
    def _():
        pltpu.sync_copy(outb,
                        out_hbm.at[pl.ds(s * NODES_PER_TILE * HID,
                                         NODES_PER_TILE * HID)])
    scope_fin.__exit__(None, None, None)


def _gcn_sc(edge_index, x16g, params):
    mesh = plsc.VectorSubcoreMesh(core_axis_name="c", subcore_axis_name="s",
                                  num_cores=1)
    f32 = jnp.float32
    kern = pl.kernel(
        _gcn_body,
        out_type=jax.ShapeDtypeStruct((N_NODES * HID,), f32),
        mesh=mesh,
        compiler_params=pltpu.CompilerParams(needs_layout_passes=False,
                                             use_tc_tiling_on_sc=False),
        scratch_types=[
            pltpu.VMEM((EDGES_PER_TILE,), jnp.int32),
            pltpu.VMEM((EDGES_PER_TILE,), jnp.int32),
            pltpu.VMEM((EDGES_PER_TILE, LANES), f32),
            pltpu.VMEM((NODES_PER_TILE, LANES), f32),
            pltpu.VMEM((NODES_PER_TILE * IN_DIM,), f32),
            pltpu.VMEM((14, LANES), f32),
            pltpu.VMEM((NODES_PER_TILE, LANES), f32),
            pltpu.VMEM((NODES_PER_TILE, LANES), f32),
            pltpu.VMEM((NODES_PER_TILE, LANES), f32),
            pltpu.VMEM((NODES_PER_TILE, LANES), f32),
            pltpu.VMEM((NODES_PER_TILE, LANES), f32),
            pltpu.VMEM((NODES_PER_TILE, LANES), f32),
            pltpu.VMEM((NODES_PER_TILE, LANES), f32),
            pltpu.VMEM((NODES_PER_TILE * HID,), f32),
            pltpu.VMEM_SHARED((N_NODES, LANES), f32),
            pltpu.VMEM_SHARED((N_NODES, LANES), f32),
            pltpu.SemaphoreType.DMA,
            pltpu.SemaphoreType.DMA,
        ],
    )
    return kern(edge_index, x16g, params)


KBLK = 512
NSLOT = 4


def _mlp_body(v_ref, w1_hbm, b1_ref, w3_ref, b3_ref, o_ref, wbuf, sems):
    n_in = w1_hbm.shape[0]
    n_hidden = w1_hbm.shape[1]
    n_slab = n_in // KBLK

    def fire(i):
        return pltpu.async_copy(
            w1_hbm.at[pl.ds(i * KBLK, KBLK), :], wbuf.at[i % NSLOT],
            sems.at[i % NSLOT])

    descs = [fire(i) for i in range(NSLOT)]
    u = jnp.zeros((1, n_hidden), jnp.float32)
    for i in range(n_slab):
        descs[i % NSLOT].wait()
        vblk = v_ref[:, pl.ds(i * KBLK, KBLK)]
        u = u + jnp.dot(vblk, wbuf[i % NSLOT],
                        preferred_element_type=jnp.float32)
        if i + NSLOT < n_slab:
            descs[i % NSLOT] = fire(i + NSLOT)
    u = jnp.maximum(u + b1_ref[...], 0.0)
    z = jnp.dot(u, w3_ref[...], preferred_element_type=jnp.float32)
    z = z + b3_ref[...]
    z = z - jnp.max(z, axis=-1, keepdims=True)
    e = jnp.exp(z)
    o_ref[...] = e / jnp.sum(e, axis=-1, keepdims=True)


def _mlp_tc(v, lin1_W, lin1_b, lin3_W, lin3_b):
    n_hidden = lin1_W.shape[1]
    n_out = lin3_W.shape[1]
    return pl.pallas_call(
        _mlp_body,
        in_specs=[
            pl.BlockSpec(memory_space=pltpu.VMEM),
            pl.BlockSpec(memory_space=pl.ANY),
            pl.BlockSpec(memory_space=pltpu.VMEM),
            pl.BlockSpec(memory_space=pltpu.VMEM),
            pl.BlockSpec(memory_space=pltpu.VMEM),
        ],
        out_specs=pl.BlockSpec(memory_space=pltpu.VMEM),
        out_shape=jax.ShapeDtypeStruct((1, n_out), jnp.float32),
        scratch_shapes=[
            pltpu.VMEM((NSLOT, KBLK, n_hidden), jnp.float32),
            pltpu.SemaphoreType.DMA((NSLOT,)),
        ],
    )(v, lin1_W, lin1_b, lin3_W, lin3_b)


def kernel(x, edge_index, W1, b1, W2, b2, lin1_W, lin1_b, lin3_W, lin3_b):
    f32 = jnp.float32
    params = (jnp.zeros((14, LANES), f32)
              .at[0:IN_DIM, :HID].set(W1)
              .at[IN_DIM, :HID].set(b1)
              .at[5:5 + HID, :HID].set(W2)
              .at[13, :HID].set(b2))
    x16g = x.reshape(N_SUB, NODES_PER_TILE * IN_DIM)
    h2 = _gcn_sc(edge_index, x16g, params)
    v = h2.reshape(1, -1)
    out = _mlp_tc(v, lin1_W, lin1_b.reshape(1, -1), lin3_W, lin3_b.reshape(1, -1))
    return out.reshape(-1)


# speedup vs baseline: 1.0998x; 1.0049x over previous
import functools

import jax
import jax.numpy as jnp
from jax import lax
from jax.experimental import pallas as pl
from jax.experimental.pallas import tpu as pltpu
from jax.experimental.pallas import tpu_sc as plsc

N_NODES = 1024
N_EDGES = 65536
IN_DIM = 4
HID = 8
LANES = 16
N_SUB = 16
NODES_PER_TILE = N_NODES // N_SUB
CHUNK = 128
ROWS_PER_TILE = (N_EDGES // CHUNK) // N_SUB
EDGES_PER_TILE = N_EDGES // N_SUB


def _rsqrt16(d):
    i = plsc.bitcast(d, jnp.int32)
    i = 0x5F3759DF - lax.shift_right_logical(i, 1)
    y = plsc.bitcast(i, jnp.float32)
    for _ in range(3):
        y = y * (1.5 - 0.5 * d * y * y)
    return y


NBUF = 8
N_ROUNDS = ROWS_PER_TILE // NBUF


def _gcn_body(edge_hbm, x_hbm, w1_hbm, b1_hbm, w2_hbm, b2_hbm,
              out_hbm,
              srcl, dstl, rowsb, onesv, xloc, pv,
              degb, dinvb, xwb, hwb, accb, pubb, zb, outb,
              t_src, t_acc, sem_g, sem_s):
    c = lax.axis_index("c")
    s = lax.axis_index("s")
    nd = pl.ds(s * NODES_PER_TILE, NODES_PER_TILE)

    scope_stage = jax.named_scope("stage")
    scope_stage.__enter__()
    stage = [
        pltpu.async_copy(
            edge_hbm.at[0, pl.ds(s * EDGES_PER_TILE, EDGES_PER_TILE)],
            srcl, sem_g),
        pltpu.async_copy(
            edge_hbm.at[1, pl.ds(s * EDGES_PER_TILE, EDGES_PER_TILE)],
            dstl, sem_g),
        pltpu.async_copy(x_hbm.at[s], xloc, sem_g),
    ]
    def fill_ones(i, carry):
        onesv[i] = jnp.full((LANES,), 1.0, jnp.float32)
        return carry
    lax.fori_loop(0, NODES_PER_TILE, fill_ones, 0)

    def fill_zero_pv(i, carry):
        pv[i] = jnp.zeros((LANES,), jnp.float32)
        return carry
    lax.fori_loop(0, 14, fill_zero_pv, 0)
    stage.append(pltpu.async_copy(w1_hbm, pv.at[pl.ds(0, IN_DIM),
                                                pl.ds(0, HID)], sem_g))
    stage.append(pltpu.async_copy(b1_hbm, pv.at[4, pl.ds(0, HID)], sem_g))
    stage.append(pltpu.async_copy(w2_hbm, pv.at[pl.ds(5, HID),
                                                pl.ds(0, HID)], sem_g))
    stage.append(pltpu.async_copy(b2_hbm, pv.at[13, pl.ds(0, HID)], sem_g))
    for d in stage:
        d.wait()
    pltpu.sync_copy(onesv, t_acc.at[nd])
    pltpu.sync_copy(onesv, t_src.at[nd])
    plsc.subcore_barrier()
    scope_stage.__exit__(None, None, None)

    scope_deg = jax.named_scope("deg_mac")
    scope_deg.__enter__()
    deg_g = pltpu.async_copy(t_src.at[srcl], rowsb, sem_g)

    w1r = [pv[k] for k in range(IN_DIM)]
    zero16 = jnp.zeros((LANES,), jnp.float32)

    def mac1(j, carry):
        xv = xloc[pl.ds(j * LANES, LANES)]
        for m in range(4):
            acc = zero16
            for k in range(IN_DIM):
                acc = acc + w1r[k] * xv[4 * m + k]
            xwb[4 * j + m] = acc
            zb[4 * j + m] = zero16
        return carry
    lax.fori_loop(0, NODES_PER_TILE // 4, mac1, 0)
    deg_g.wait()
    pltpu.async_copy(rowsb, t_acc.at[dstl], sem_s, add=True).wait()
    plsc.subcore_barrier()
    scope_deg.__exit__(None, None, None)

    scope_pub = jax.named_scope("dinv_pub")
    scope_pub.__enter__()
    pltpu.sync_copy(t_acc.at[nd], degb)
    b1 = pv[4]
    b2 = pv[13]
    w2r = [pv[5 + k] for k in range(HID)]

    def loc1(i, carry):
        y = _rsqrt16(degb[i])
        dinvb[i] = y
        pubb[i] = xwb[i] * y
        return carry
    lax.fori_loop(0, NODES_PER_TILE, loc1, 0)
    pltpu.sync_copy(pubb, t_src.at[nd])
    pltpu.sync_copy(zb, t_acc.at[nd])
    plsc.subcore_barrier()
    scope_pub.__exit__(None, None, None)

    HALF = EDGES_PER_TILE // 2

    def edge_pass():
        ga = pltpu.async_copy(t_src.at[srcl.at[pl.ds(0, HALF)]],
                              rowsb.at[pl.ds(0, HALF)], sem_g)
        ga.wait()
        sa = pltpu.async_copy(rowsb.at[pl.ds(0, HALF)],
                              t_acc.at[dstl.at[pl.ds(0, HALF)]],
                              sem_s, add=True)
        gb = pltpu.async_copy(t_src.at[srcl.at[pl.ds(HALF, HALF)]],
                              rowsb.at[pl.ds(HALF, HALF)], sem_g)
        gb.wait()
        sa.wait()
        pltpu.async_copy(rowsb.at[pl.ds(HALF, HALF)],
                         t_acc.at[dstl.at[pl.ds(HALF, HALF)]],
                         sem_s, add=True).wait()

    with jax.named_scope("l1_edges"):
        edge_pass()
    plsc.subcore_barrier()

    scope_mid = jax.named_scope("mid_locals")
    scope_mid.__enter__()
    pltpu.sync_copy(t_acc.at[nd], accb)

    def loc2(i, carry):
        y = dinvb[i]
        h1 = jnp.maximum(y * accb[i] + (y * y) * xwb[i] + b1, 0.0)
        acc = jnp.zeros((LANES,), jnp.float32)
        for k in range(HID):
            acc = acc + w2r[k] * h1[k]
        hwb[i] = acc
        pubb[i] = acc * y
        return carry
    lax.fori_loop(0, NODES_PER_TILE, loc2, 0)
    pltpu.sync_copy(pubb, t_src.at[nd])
    pltpu.sync_copy(zb, t_acc.at[nd])
    plsc.subcore_barrier()
    scope_mid.__exit__(None, None, None)

    with jax.named_scope("l2_edges"):
        edge_pass()
    plsc.subcore_barrier()

    scope_fin = jax.named_scope("final")
    scope_fin.__enter__()
    pltpu.sync_copy(t_acc.at[nd], accb)

    lanes = lax.iota(jnp.int32, LANES)
    lo = lanes < HID

    def loc4(i, carry):
        y = dinvb[i]
        o = y * accb[i] + (y * y) * hwb[i] + b2
        plsc.store_scatter(outb, [i * HID + lanes], o, mask=lo)
        return carry
    lax.fori_loop(0, NODES_PER_TILE, loc4, 0)

    @pl.when(c == 0)
    def _():
        pltpu.sync_copy(outb,
                        out_hbm.at[pl.ds(s * NODES_PER_TILE * HID,
                                         NODES_PER_TILE * HID)])
    scope_fin.__exit__(None, None, None)


def _gcn_sc(edge_index, x16g, W1, b1, W2, b2):
    mesh = plsc.VectorSubcoreMesh(core_axis_name="c", subcore_axis_name="s",
                                  num_cores=1)
    f32 = jnp.float32
    kern = pl.kernel(
        _gcn_body,
        out_type=jax.ShapeDtypeStruct((N_NODES * HID,), f32),
        mesh=mesh,
        compiler_params=pltpu.CompilerParams(needs_layout_passes=False,
                                             use_tc_tiling_on_sc=False),
        scratch_types=[
            pltpu.VMEM((EDGES_PER_TILE,), jnp.int32),
            pltpu.VMEM((EDGES_PER_TILE,), jnp.int32),
            pltpu.VMEM((EDGES_PER_TILE, LANES), f32),
            pltpu.VMEM((NODES_PER_TILE, LANES), f32),
            pltpu.VMEM((NODES_PER_TILE * IN_DIM,), f32),
            pltpu.VMEM((14, LANES), f32),
            pltpu.VMEM((NODES_PER_TILE, LANES), f32),
            pltpu.VMEM((NODES_PER_TILE, LANES), f32),
            pltpu.VMEM((NODES_PER_TILE, LANES), f32),
            pltpu.VMEM((NODES_PER_TILE, LANES), f32),
            pltpu.VMEM((NODES_PER_TILE, LANES), f32),
            pltpu.VMEM((NODES_PER_TILE, LANES), f32),
            pltpu.VMEM((NODES_PER_TILE, LANES), f32),
            pltpu.VMEM((NODES_PER_TILE * HID,), f32),
            pltpu.VMEM_SHARED((N_NODES, LANES), f32),
            pltpu.VMEM_SHARED((N_NODES, LANES), f32),
            pltpu.SemaphoreType.DMA,
            pltpu.SemaphoreType.DMA,
        ],
    )
    return kern(edge_index, x16g, W1, b1, W2, b2)


KBLK = 512
NSLOT = 5


def _mlp_body(v_ref, w1_hbm, b1_ref, w3_ref, b3_ref, o_ref, wbuf, sems):
    n_in = w1_hbm.shape[0]
    n_hidden = w1_hbm.shape[1]
    n_slab = n_in // KBLK

    def fire(i):
        return pltpu.async_copy(
            w1_hbm.at[pl.ds(i * KBLK, KBLK), :], wbuf.at[i % NSLOT],
            sems.at[i % NSLOT])

    descs = [fire(i) for i in range(NSLOT)]
    u = jnp.zeros((1, n_hidden), jnp.float32)
    for i in range(n_slab):
        descs[i % NSLOT].wait()
        vblk = v_ref[:, pl.ds(i * KBLK, KBLK)]
        u = u + jnp.dot(vblk, wbuf[i % NSLOT],
                        preferred_element_type=jnp.float32)
        if i + NSLOT < n_slab:
            descs[i % NSLOT] = fire(i + NSLOT)
    u = jnp.maximum(u + b1_ref[...], 0.0)
    z = jnp.dot(u, w3_ref[...], preferred_element_type=jnp.float32)
    z = z + b3_ref[...]
    z = z - jnp.max(z, axis=-1, keepdims=True)
    e = jnp.exp(z)
    o_ref[...] = e / jnp.sum(e, axis=-1, keepdims=True)


def _mlp_tc(v, lin1_W, lin1_b, lin3_W, lin3_b):
    n_hidden = lin1_W.shape[1]
    n_out = lin3_W.shape[1]
    return pl.pallas_call(
        _mlp_body,
        in_specs=[
            pl.BlockSpec(memory_space=pltpu.VMEM),
            pl.BlockSpec(memory_space=pl.ANY),
            pl.BlockSpec(memory_space=pltpu.VMEM),
            pl.BlockSpec(memory_space=pltpu.VMEM),
            pl.BlockSpec(memory_space=pltpu.VMEM),
        ],
        out_specs=pl.BlockSpec(memory_space=pltpu.VMEM),
        out_shape=jax.ShapeDtypeStruct((1, n_out), jnp.float32),
        scratch_shapes=[
            pltpu.VMEM((NSLOT, KBLK, n_hidden), jnp.float32),
            pltpu.SemaphoreType.DMA((NSLOT,)),
        ],
    )(v, lin1_W, lin1_b, lin3_W, lin3_b)


def kernel(x, edge_index, W1, b1, W2, b2, lin1_W, lin1_b, lin3_W, lin3_b):
    x16g = x.reshape(N_SUB, NODES_PER_TILE * IN_DIM)
    h2 = _gcn_sc(edge_index, x16g, W1, b1, W2, b2)
    v = h2.reshape(1, -1)
    out = _mlp_tc(v, lin1_W, lin1_b.reshape(1, -1), lin3_W, lin3_b.reshape(1, -1))
    return out.reshape(-1)


# speedup vs baseline: 1.1118x; 1.0109x over previous
import functools

import jax
import jax.numpy as jnp
from jax import lax
from jax.experimental import pallas as pl
from jax.experimental.pallas import tpu as pltpu
from jax.experimental.pallas import tpu_sc as plsc

N_NODES = 1024
N_EDGES = 65536
IN_DIM = 4
HID = 8
LANES = 16
N_SUB = 16
NODES_PER_TILE = N_NODES // N_SUB
CHUNK = 128
ROWS_PER_TILE = (N_EDGES // CHUNK) // N_SUB
EDGES_PER_TILE = N_EDGES // N_SUB


def _rsqrt16(d):
    i = plsc.bitcast(d, jnp.int32)
    i = 0x5F3759DF - lax.shift_right_logical(i, 1)
    y = plsc.bitcast(i, jnp.float32)
    for _ in range(3):
        y = y * (1.5 - 0.5 * d * y * y)
    return y


NBUF = 8
N_ROUNDS = ROWS_PER_TILE // NBUF


def _gcn_body(edge_hbm, x_hbm, w1_hbm, b1_hbm, w2_hbm, b2_hbm,
              out_hbm,
              srcl, dstl, rowsb, onesv, xloc, pv,
              degb, dinvb, xwb, hwb, accb, pubb, zb, outb,
              t_src, t_acc, sem_g, sem_s):
    c = lax.axis_index("c")
    s = lax.axis_index("s")
    nd = pl.ds(s * NODES_PER_TILE, NODES_PER_TILE)

    scope_stage = jax.named_scope("stage")
    scope_stage.__enter__()
    stage = [
        pltpu.async_copy(
            edge_hbm.at[0, pl.ds(s * EDGES_PER_TILE, EDGES_PER_TILE)],
            srcl, sem_g),
        pltpu.async_copy(
            edge_hbm.at[1, pl.ds(s * EDGES_PER_TILE, EDGES_PER_TILE)],
            dstl, sem_g),
        pltpu.async_copy(x_hbm.at[s], xloc, sem_g),
    ]
    def fill_ones(i, carry):
        onesv[i] = jnp.full((LANES,), 1.0, jnp.float32)
        return carry
    lax.fori_loop(0, NODES_PER_TILE, fill_ones, 0)

    def fill_zero_pv(i, carry):
        pv[i] = jnp.zeros((LANES,), jnp.float32)
        return carry
    lax.fori_loop(0, 14, fill_zero_pv, 0)
    stage.append(pltpu.async_copy(w1_hbm, pv.at[pl.ds(0, IN_DIM),
                                                pl.ds(0, HID)], sem_g))
    stage.append(pltpu.async_copy(b1_hbm, pv.at[4, pl.ds(0, HID)], sem_g))
    stage.append(pltpu.async_copy(w2_hbm, pv.at[pl.ds(5, HID),
                                                pl.ds(0, HID)], sem_g))
    stage.append(pltpu.async_copy(b2_hbm, pv.at[13, pl.ds(0, HID)], sem_g))
    for d in stage:
        d.wait()
    pltpu.sync_copy(onesv, t_acc.at[nd])
    pltpu.sync_copy(onesv, t_src.at[nd])
    plsc.subcore_barrier()
    scope_stage.__exit__(None, None, None)

    scope_deg = jax.named_scope("deg_mac")
    scope_deg.__enter__()
    deg_g = pltpu.async_copy(t_src.at[srcl], rowsb, sem_g)

    w1r = [pv[k] for k in range(IN_DIM)]
    zero16 = jnp.zeros((LANES,), jnp.float32)

    def mac1(j, carry):
        xv = xloc[pl.ds(j * LANES, LANES)]
        for m in range(4):
            acc = zero16
            for k in range(IN_DIM):
                acc = acc + w1r[k] * xv[4 * m + k]
            xwb[4 * j + m] = acc
            zb[4 * j + m] = zero16
        return carry
    lax.fori_loop(0, NODES_PER_TILE // 4, mac1, 0)
    deg_g.wait()
    pltpu.async_copy(rowsb, t_acc.at[dstl], sem_s, add=True).wait()
    plsc.subcore_barrier()
    scope_deg.__exit__(None, None, None)

    scope_pub = jax.named_scope("dinv_pub")
    scope_pub.__enter__()
    pltpu.sync_copy(t_acc.at[nd], degb)
    b1 = pv[4]
    b2 = pv[13]
    w2r = [pv[5 + k] for k in range(HID)]

    def loc1(i, carry):
        y = _rsqrt16(degb[i])
        dinvb[i] = y
        pubb[i] = xwb[i] * y
        return carry
    lax.fori_loop(0, NODES_PER_TILE, loc1, 0)
    pltpu.sync_copy(pubb, t_src.at[nd])
    pltpu.sync_copy(zb, t_acc.at[nd])
    plsc.subcore_barrier()
    scope_pub.__exit__(None, None, None)

    HALF = EDGES_PER_TILE // 2

    def edge_pass():
        ga = pltpu.async_copy(t_src.at[srcl.at[pl.ds(0, HALF)]],
                              rowsb.at[pl.ds(0, HALF)], sem_g)
        ga.wait()
        sa = pltpu.async_copy(rowsb.at[pl.ds(0, HALF)],
                              t_acc.at[dstl.at[pl.ds(0, HALF)]],
                              sem_s, add=True)
        gb = pltpu.async_copy(t_src.at[srcl.at[pl.ds(HALF, HALF)]],
                              rowsb.at[pl.ds(HALF, HALF)], sem_g)
        gb.wait()
        sa.wait()
        pltpu.async_copy(rowsb.at[pl.ds(HALF, HALF)],
                         t_acc.at[dstl.at[pl.ds(HALF, HALF)]],
                         sem_s, add=True).wait()

    with jax.named_scope("l1_edges"):
        edge_pass()
    plsc.subcore_barrier()

    scope_mid = jax.named_scope("mid_locals")
    scope_mid.__enter__()
    pltpu.sync_copy(t_acc.at[nd], accb)

    def loc2(i, carry):
        y = dinvb[i]
        h1 = jnp.maximum(y * accb[i] + (y * y) * xwb[i] + b1, 0.0)
        acc = jnp.zeros((LANES,), jnp.float32)
        for k in range(HID):
            acc = acc + w2r[k] * h1[k]
        hwb[i] = acc
        pubb[i] = acc * y
        return carry
    lax.fori_loop(0, NODES_PER_TILE, loc2, 0)
    pltpu.sync_copy(pubb, t_src.at[nd])
    pltpu.sync_copy(zb, t_acc.at[nd])
    plsc.subcore_barrier()
    scope_mid.__exit__(None, None, None)

    with jax.named_scope("l2_edges"):
        edge_pass()
    plsc.subcore_barrier()

    scope_fin = jax.named_scope("final")
    scope_fin.__enter__()
    pltpu.sync_copy(t_acc.at[nd], accb)

    lanes = lax.iota(jnp.int32, LANES)
    lo = lanes < HID

    def loc4(i, carry):
        y = dinvb[i]
        o = y * accb[i] + (y * y) * hwb[i] + b2
        plsc.store_scatter(outb, [i * HID + lanes], o, mask=lo)
        return carry
    lax.fori_loop(0, NODES_PER_TILE, loc4, 0)

    @pl.when(c == 0)
    def _():
        pltpu.sync_copy(outb,
                        out_hbm.at[pl.ds(s * NODES_PER_TILE * HID,
                                         NODES_PER_TILE * HID)])
    scope_fin.__exit__(None, None, None)


def _gcn_sc(edge_index, x16g, W1, b1, W2, b2):
    mesh = plsc.VectorSubcoreMesh(core_axis_name="c", subcore_axis_name="s",
                                  num_cores=1)
    f32 = jnp.float32
    kern = pl.kernel(
        _gcn_body,
        out_type=jax.ShapeDtypeStruct((N_NODES * HID,), f32),
        mesh=mesh,
        compiler_params=pltpu.CompilerParams(needs_layout_passes=False,
                                             use_tc_tiling_on_sc=False),
        scratch_types=[
            pltpu.VMEM((EDGES_PER_TILE,), jnp.int32),
            pltpu.VMEM((EDGES_PER_TILE,), jnp.int32),
            pltpu.VMEM((EDGES_PER_TILE, LANES), f32),
            pltpu.VMEM((NODES_PER_TILE, LANES), f32),
            pltpu.VMEM((NODES_PER_TILE * IN_DIM,), f32),
            pltpu.VMEM((14, LANES), f32),
            pltpu.VMEM((NODES_PER_TILE, LANES), f32),
            pltpu.VMEM((NODES_PER_TILE, LANES), f32),
            pltpu.VMEM((NODES_PER_TILE, LANES), f32),
            pltpu.VMEM((NODES_PER_TILE, LANES), f32),
            pltpu.VMEM((NODES_PER_TILE, LANES), f32),
            pltpu.VMEM((NODES_PER_TILE, LANES), f32),
            pltpu.VMEM((NODES_PER_TILE, LANES), f32),
            pltpu.VMEM((NODES_PER_TILE * HID,), f32),
            pltpu.VMEM_SHARED((N_NODES, LANES), f32),
            pltpu.VMEM_SHARED((N_NODES, LANES), f32),
            pltpu.SemaphoreType.DMA,
            pltpu.SemaphoreType.DMA,
        ],
    )
    return kern(edge_index, x16g, W1, b1, W2, b2)


KBLK = 512
NSLOT = 6


def _mlp_body(v_ref, w1_hbm, b1_ref, w3_ref, b3_ref, o_ref, wbuf, sems):
    n_in = w1_hbm.shape[0]
    n_hidden = w1_hbm.shape[1]
    n_slab = n_in // KBLK

    def fire(i):
        return pltpu.async_copy(
            w1_hbm.at[pl.ds(i * KBLK, KBLK), :], wbuf.at[i % NSLOT],
            sems.at[i % NSLOT])

    descs = [fire(i) for i in range(NSLOT)]
    u = jnp.zeros((1, n_hidden), jnp.float32)
    for i in range(n_slab):
        descs[i % NSLOT].wait()
        vblk = v_ref[:, pl.ds(i * KBLK, KBLK)]
        u = u + jnp.dot(vblk, wbuf[i % NSLOT],
                        preferred_element_type=jnp.float32)
        if i + NSLOT < n_slab:
            descs[i % NSLOT] = fire(i + NSLOT)
    u = jnp.maximum(u + b1_ref[...], 0.0)
    z = jnp.dot(u, w3_ref[...], preferred_element_type=jnp.float32)
    z = z + b3_ref[...]
    z = z - jnp.max(z, axis=-1, keepdims=True)
    e = jnp.exp(z)
    o_ref[...] = e / jnp.sum(e, axis=-1, keepdims=True)


def _mlp_tc(v, lin1_W, lin1_b, lin3_W, lin3_b):
    n_hidden = lin1_W.shape[1]
    n_out = lin3_W.shape[1]
    return pl.pallas_call(
        _mlp_body,
        in_specs=[
            pl.BlockSpec(memory_space=pltpu.VMEM),
            pl.BlockSpec(memory_space=pl.ANY),
            pl.BlockSpec(memory_space=pltpu.VMEM),
            pl.BlockSpec(memory_space=pltpu.VMEM),
            pl.BlockSpec(memory_space=pltpu.VMEM),
        ],
        out_specs=pl.BlockSpec(memory_space=pltpu.VMEM),
        out_shape=jax.ShapeDtypeStruct((1, n_out), jnp.float32),
        compiler_params=pltpu.CompilerParams(
            vmem_limit_bytes=64 * 1024 * 1024),
        scratch_shapes=[
            pltpu.VMEM((NSLOT, KBLK, n_hidden), jnp.float32),
            pltpu.SemaphoreType.DMA((NSLOT,)),
        ],
    )(v, lin1_W, lin1_b, lin3_W, lin3_b)


def kernel(x, edge_index, W1, b1, W2, b2, lin1_W, lin1_b, lin3_W, lin3_b):
    x16g = x.reshape(N_SUB, NODES_PER_TILE * IN_DIM)
    h2 = _gcn_sc(edge_index, x16g, W1, b1, W2, b2)
    v = h2.reshape(1, -1)
    out = _mlp_tc(v, lin1_W, lin1_b.reshape(1, -1), lin3_W, lin3_b.reshape(1, -1))
    return out.reshape(-1)


# speedup vs baseline: 1.1272x; 1.0139x over previous
import functools

import jax
import jax.numpy as jnp
from jax import lax
from jax.experimental import pallas as pl
from jax.experimental.pallas import tpu as pltpu
from jax.experimental.pallas import tpu_sc as plsc

N_NODES = 1024
N_EDGES = 65536
IN_DIM = 4
HID = 8
LANES = 16
N_SUB = 16
NODES_PER_TILE = N_NODES // N_SUB
CHUNK = 128
ROWS_PER_TILE = (N_EDGES // CHUNK) // N_SUB
EDGES_PER_TILE = N_EDGES // N_SUB


def _rsqrt16(d):
    i = plsc.bitcast(d, jnp.int32)
    i = 0x5F3759DF - lax.shift_right_logical(i, 1)
    y = plsc.bitcast(i, jnp.float32)
    for _ in range(3):
        y = y * (1.5 - 0.5 * d * y * y)
    return y


NBUF = 8
N_ROUNDS = ROWS_PER_TILE // NBUF


def _gcn_body(edge_hbm, x_hbm, w1_hbm, b1_hbm, w2_hbm, b2_hbm,
              out_hbm,
              srcl, dstl, rowsb, onesv, xloc, pv,
              degb, dinvb, xwb, hwb, accb, pubb, zb, outb,
              t_src, t_acc, sem_g, sem_s):
    c = lax.axis_index("c")
    s = lax.axis_index("s")
    nd = pl.ds(s * NODES_PER_TILE, NODES_PER_TILE)

    scope_stage = jax.named_scope("stage")
    scope_stage.__enter__()
    stage = [
        pltpu.async_copy(
            edge_hbm.at[0, pl.ds(s * EDGES_PER_TILE, EDGES_PER_TILE)],
            srcl, sem_g),
        pltpu.async_copy(
            edge_hbm.at[1, pl.ds(s * EDGES_PER_TILE, EDGES_PER_TILE)],
            dstl, sem_g),
        pltpu.async_copy(x_hbm.at[s], xloc, sem_g),
    ]
    def fill_ones(i, carry):
        onesv[i] = jnp.full((LANES,), 1.0, jnp.float32)
        return carry
    lax.fori_loop(0, NODES_PER_TILE, fill_ones, 0)

    def fill_zero_pv(i, carry):
        pv[i] = jnp.zeros((LANES,), jnp.float32)
        return carry
    lax.fori_loop(0, 14, fill_zero_pv, 0)
    stage.append(pltpu.async_copy(w1_hbm, pv.at[pl.ds(0, IN_DIM),
                                                pl.ds(0, HID)], sem_g))
    stage.append(pltpu.async_copy(b1_hbm, pv.at[4, pl.ds(0, HID)], sem_g))
    stage.append(pltpu.async_copy(w2_hbm, pv.at[pl.ds(5, HID),
                                                pl.ds(0, HID)], sem_g))
    stage.append(pltpu.async_copy(b2_hbm, pv.at[13, pl.ds(0, HID)], sem_g))
    for d in stage:
        d.wait()
    pltpu.sync_copy(onesv, t_acc.at[nd])
    plsc.subcore_barrier()
    scope_stage.__exit__(None, None, None)

    scope_deg = jax.named_scope("deg_mac")
    scope_deg.__enter__()
    deg_ds = [pltpu.async_copy(onesv,
                               t_acc.at[dstl.at[pl.ds(j * NODES_PER_TILE,
                                                      NODES_PER_TILE)]],
                               sem_s, add=True)
              for j in range(EDGES_PER_TILE // NODES_PER_TILE)]

    w1r = [pv[k] for k in range(IN_DIM)]
    zero16 = jnp.zeros((LANES,), jnp.float32)

    def mac1(j, carry):
        xv = xloc[pl.ds(j * LANES, LANES)]
        for m in range(4):
            acc = zero16
            for k in range(IN_DIM):
                acc = acc + w1r[k] * xv[4 * m + k]
            xwb[4 * j + m] = acc
            zb[4 * j + m] = zero16
        return carry
    lax.fori_loop(0, NODES_PER_TILE // 4, mac1, 0)
    for d in deg_ds:
        d.wait()
    plsc.subcore_barrier()
    scope_deg.__exit__(None, None, None)

    scope_pub = jax.named_scope("dinv_pub")
    scope_pub.__enter__()
    pltpu.sync_copy(t_acc.at[nd], degb)
    b1 = pv[4]
    b2 = pv[13]
    w2r = [pv[5 + k] for k in range(HID)]

    def loc1(i, carry):
        y = _rsqrt16(degb[i])
        dinvb[i] = y
        pubb[i] = xwb[i] * y
        return carry
    lax.fori_loop(0, NODES_PER_TILE, loc1, 0)
    pltpu.sync_copy(pubb, t_src.at[nd])
    pltpu.sync_copy(zb, t_acc.at[nd])
    plsc.subcore_barrier()
    scope_pub.__exit__(None, None, None)

    HALF = EDGES_PER_TILE // 2

    def edge_pass():
        ga = pltpu.async_copy(t_src.at[srcl.at[pl.ds(0, HALF)]],
                              rowsb.at[pl.ds(0, HALF)], sem_g)
        ga.wait()
        sa = pltpu.async_copy(rowsb.at[pl.ds(0, HALF)],
                              t_acc.at[dstl.at[pl.ds(0, HALF)]],
                              sem_s, add=True)
        gb = pltpu.async_copy(t_src.at[srcl.at[pl.ds(HALF, HALF)]],
                              rowsb.at[pl.ds(HALF, HALF)], sem_g)
        gb.wait()
        sa.wait()
        pltpu.async_copy(rowsb.at[pl.ds(HALF, HALF)],
                         t_acc.at[dstl.at[pl.ds(HALF, HALF)]],
                         sem_s, add=True).wait()

    with jax.named_scope("l1_edges"):
        edge_pass()
    plsc.subcore_barrier()

    scope_mid = jax.named_scope("mid_locals")
    scope_mid.__enter__()
    pltpu.sync_copy(t_acc.at[nd], accb)

    def loc2(i, carry):
        y = dinvb[i]
        h1 = jnp.maximum(y * accb[i] + (y * y) * xwb[i] + b1, 0.0)
        acc = jnp.zeros((LANES,), jnp.float32)
        for k in range(HID):
            acc = acc + w2r[k] * h1[k]
        hwb[i] = acc
        pubb[i] = acc * y
        return carry
    lax.fori_loop(0, NODES_PER_TILE, loc2, 0)
    pltpu.sync_copy(pubb, t_src.at[nd])
    pltpu.sync_copy(zb, t_acc.at[nd])
    plsc.subcore_barrier()
    scope_mid.__exit__(None, None, None)

    with jax.named_scope("l2_edges"):
        edge_pass()
    plsc.subcore_barrier()

    scope_fin = jax.named_scope("final")
    scope_fin.__enter__()
    pltpu.sync_copy(t_acc.at[nd], accb)

    lanes = lax.iota(jnp.int32, LANES)
    lo = lanes < HID

    def loc4(i, carry):
        y = dinvb[i]
        o = y * accb[i] + (y * y) * hwb[i] + b2
        plsc.store_scatter(outb, [i * HID + lanes], o, mask=lo)
        return carry
    lax.fori_loop(0, NODES_PER_TILE, loc4, 0)

    @pl.when(c == 0)
    def _():
        pltpu.sync_copy(outb,
                        out_hbm.at[pl.ds(s * NODES_PER_TILE * HID,
                                         NODES_PER_TILE * HID)])
    scope_fin.__exit__(None, None, None)


def _gcn_sc(edge_index, x16g, W1, b1, W2, b2):
    mesh = plsc.VectorSubcoreMesh(core_axis_name="c", subcore_axis_name="s",
                                  num_cores=1)
    f32 = jnp.float32
    kern = pl.kernel(
        _gcn_body,
        out_type=jax.ShapeDtypeStruct((N_NODES * HID,), f32),
        mesh=mesh,
        compiler_params=pltpu.CompilerParams(needs_layout_passes=False,
                                             use_tc_tiling_on_sc=False),
        scratch_types=[
            pltpu.VMEM((EDGES_PER_TILE,), jnp.int32),
            pltpu.VMEM((EDGES_PER_TILE,), jnp.int32),
            pltpu.VMEM((EDGES_PER_TILE, LANES), f32),
            pltpu.VMEM((NODES_PER_TILE, LANES), f32),
            pltpu.VMEM((NODES_PER_TILE * IN_DIM,), f32),
            pltpu.VMEM((14, LANES), f32),
            pltpu.VMEM((NODES_PER_TILE, LANES), f32),
            pltpu.VMEM((NODES_PER_TILE, LANES), f32),
            pltpu.VMEM((NODES_PER_TILE, LANES), f32),
            pltpu.VMEM((NODES_PER_TILE, LANES), f32),
            pltpu.VMEM((NODES_PER_TILE, LANES), f32),
            pltpu.VMEM((NODES_PER_TILE, LANES), f32),
            pltpu.VMEM((NODES_PER_TILE, LANES), f32),
            pltpu.VMEM((NODES_PER_TILE * HID,), f32),
            pltpu.VMEM_SHARED((N_NODES, LANES), f32),
            pltpu.VMEM_SHARED((N_NODES, LANES), f32),
            pltpu.SemaphoreType.DMA,
            pltpu.SemaphoreType.DMA,
        ],
    )
    return kern(edge_index, x16g, W1, b1, W2, b2)


KBLK = 512
NSLOT = 6


def _mlp_body(v_ref, w1_hbm, b1_ref, w3_ref, b3_ref, o_ref, wbuf, sems):
    n_in = w1_hbm.shape[0]
    n_hidden = w1_hbm.shape[1]
    n_slab = n_in // KBLK

    def fire(i):
        return pltpu.async_copy(
            w1_hbm.at[pl.ds(i * KBLK, KBLK), :], wbuf.at[i % NSLOT],
            sems.at[i % NSLOT])

    descs = [fire(i) for i in range(NSLOT)]
    u = jnp.zeros((1, n_hidden), jnp.float32)
    for i in range(n_slab):
        descs[i % NSLOT].wait()
        vblk = v_ref[:, pl.ds(i * KBLK, KBLK)]
        u = u + jnp.dot(vblk, wbuf[i % NSLOT],
                        preferred_element_type=jnp.float32)
        if i + NSLOT < n_slab:
            descs[i % NSLOT] = fire(i + NSLOT)
    u = jnp.maximum(u + b1_ref[...], 0.0)
    z = jnp.dot(u, w3_ref[...], preferred_element_type=jnp.float32)
    z = z + b3_ref[...]
    z = z - jnp.max(z, axis=-1, keepdims=True)
    e = jnp.exp(z)
    o_ref[...] = e / jnp.sum(e, axis=-1, keepdims=True)


def _mlp_tc(v, lin1_W, lin1_b, lin3_W, lin3_b):
    n_hidden = lin1_W.shape[1]
    n_out = lin3_W.shape[1]
    return pl.pallas_call(
        _mlp_body,
        in_specs=[
            pl.BlockSpec(memory_space=pltpu.VMEM),
            pl.BlockSpec(memory_space=pl.ANY),
            pl.BlockSpec(memory_space=pltpu.VMEM),
            pl.BlockSpec(memory_space=pltpu.VMEM),
            pl.BlockSpec(memory_space=pltpu.VMEM),
        ],
        out_specs=pl.BlockSpec(memory_space=pltpu.VMEM),
        out_shape=jax.ShapeDtypeStruct((1, n_out), jnp.float32),
        compiler_params=pltpu.CompilerParams(
            vmem_limit_bytes=64 * 1024 * 1024),
        scratch_shapes=[
            pltpu.VMEM((NSLOT, KBLK, n_hidden), jnp.float32),
            pltpu.SemaphoreType.DMA((NSLOT,)),
        ],
    )(v, lin1_W, lin1_b, lin3_W, lin3_b)


def kernel(x, edge_index, W1, b1, W2, b2, lin1_W, lin1_b, lin3_W, lin3_b):
    x16g = x.reshape(N_SUB, NODES_PER_TILE * IN_DIM)
    h2 = _gcn_sc(edge_index, x16g, W1, b1, W2, b2)
    v = h2.reshape(1, -1)
    out = _mlp_tc(v, lin1_W, lin1_b.reshape(1, -1), lin3_W, lin3_b.reshape(1, -1))
    return out.reshape(-1)


# speedup vs baseline: 1.1286x; 1.0012x over previous
import jax
import jax.numpy as jnp
from jax import lax
from jax.experimental import pallas as pl
from jax.experimental.pallas import tpu as pltpu
from jax.experimental.pallas import tpu_sc as plsc

N_NODES = 1024
N_EDGES = 65536
IN_DIM = 4
HID = 8
LANES = 16
N_SUB = 16
NODES_PER_TILE = N_NODES // N_SUB
EDGES_PER_TILE = N_EDGES // N_SUB


def _rsqrt16(d):
    i = plsc.bitcast(d, jnp.int32)
    i = 0x5F3759DF - lax.shift_right_logical(i, 1)
    y = plsc.bitcast(i, jnp.float32)
    for _ in range(3):
        y = y * (1.5 - 0.5 * d * y * y)
    return y


def _gcn_body(edge_hbm, x_hbm, w1_hbm, b1_hbm, w2_hbm, b2_hbm,
              out_hbm,
              srcl, dstl, rowsb, onesv, xloc, pv,
              degb, dinvb, xwb, hwb, accb, pubb, zb, outb,
              t_src, t_acc, sem_g, sem_s):
    c = lax.axis_index("c")
    s = lax.axis_index("s")
    nd = pl.ds(s * NODES_PER_TILE, NODES_PER_TILE)

    scope_stage = jax.named_scope("stage")
    scope_stage.__enter__()
    stage = [
        pltpu.async_copy(
            edge_hbm.at[0, pl.ds(s * EDGES_PER_TILE, EDGES_PER_TILE)],
            srcl, sem_g),
        pltpu.async_copy(
            edge_hbm.at[1, pl.ds(s * EDGES_PER_TILE, EDGES_PER_TILE)],
            dstl, sem_g),
        pltpu.async_copy(x_hbm.at[s], xloc, sem_g),
    ]
    def fill_ones(i, carry):
        onesv[i] = jnp.full((LANES,), 1.0, jnp.float32)
        return carry
    lax.fori_loop(0, NODES_PER_TILE, fill_ones, 0)

    def fill_zero_pv(i, carry):
        pv[i] = jnp.zeros((LANES,), jnp.float32)
        return carry
    lax.fori_loop(0, 14, fill_zero_pv, 0)
    stage.append(pltpu.async_copy(w1_hbm, pv.at[pl.ds(0, IN_DIM),
                                                pl.ds(0, HID)], sem_g))
    stage.append(pltpu.async_copy(b1_hbm, pv.at[4, pl.ds(0, HID)], sem_g))
    stage.append(pltpu.async_copy(w2_hbm, pv.at[pl.ds(5, HID),
                                                pl.ds(0, HID)], sem_g))
    stage.append(pltpu.async_copy(b2_hbm, pv.at[13, pl.ds(0, HID)], sem_g))
    for d in stage:
        d.wait()
    pltpu.sync_copy(onesv, t_acc.at[nd])
    plsc.subcore_barrier()
    scope_stage.__exit__(None, None, None)

    scope_deg = jax.named_scope("deg_mac")
    scope_deg.__enter__()
    deg_ds = [pltpu.async_copy(onesv,
                               t_acc.at[dstl.at[pl.ds(j * NODES_PER_TILE,
                                                      NODES_PER_TILE)]],
                               sem_s, add=True)
              for j in range(EDGES_PER_TILE // NODES_PER_TILE)]

    w1r = [pv[k] for k in range(IN_DIM)]
    zero16 = jnp.zeros((LANES,), jnp.float32)

    def mac1(j, carry):
        xv = xloc[pl.ds(j * LANES, LANES)]
        for m in range(4):
            acc = zero16
            for k in range(IN_DIM):
                acc = acc + w1r[k] * xv[4 * m + k]
            xwb[4 * j + m] = acc
            zb[4 * j + m] = zero16
        return carry
    lax.fori_loop(0, NODES_PER_TILE // 4, mac1, 0)
    for d in deg_ds:
        d.wait()
    plsc.subcore_barrier()
    scope_deg.__exit__(None, None, None)

    scope_pub = jax.named_scope("dinv_pub")
    scope_pub.__enter__()
    pltpu.sync_copy(t_acc.at[nd], degb)
    b1 = pv[4]
    b2 = pv[13]
    w2r = [pv[5 + k] for k in range(HID)]

    def loc1(i, carry):
        y = _rsqrt16(degb[i])
        dinvb[i] = y
        pubb[i] = xwb[i] * y
        return carry
    lax.fori_loop(0, NODES_PER_TILE, loc1, 0)
    pltpu.sync_copy(pubb, t_src.at[nd])
    pltpu.sync_copy(zb, t_acc.at[nd])
    plsc.subcore_barrier()
    scope_pub.__exit__(None, None, None)

    HALF = EDGES_PER_TILE // 2

    def edge_pass():
        ga = pltpu.async_copy(t_src.at[srcl.at[pl.ds(0, HALF)]],
                              rowsb.at[pl.ds(0, HALF)], sem_g)
        ga.wait()
        sa = pltpu.async_copy(rowsb.at[pl.ds(0, HALF)],
                              t_acc.at[dstl.at[pl.ds(0, HALF)]],
                              sem_s, add=True)
        gb = pltpu.async_copy(t_src.at[srcl.at[pl.ds(HALF, HALF)]],
                              rowsb.at[pl.ds(HALF, HALF)], sem_g)
        gb.wait()
        sa.wait()
        pltpu.async_copy(rowsb.at[pl.ds(HALF, HALF)],
                         t_acc.at[dstl.at[pl.ds(HALF, HALF)]],
                         sem_s, add=True).wait()

    with jax.named_scope("l1_edges"):
        edge_pass()
    plsc.subcore_barrier()

    scope_mid = jax.named_scope("mid_locals")
    scope_mid.__enter__()
    pltpu.sync_copy(t_acc.at[nd], accb)

    def loc2(i, carry):
        y = dinvb[i]
        h1 = jnp.maximum(y * accb[i] + (y * y) * xwb[i] + b1, 0.0)
        acc = jnp.zeros((LANES,), jnp.float32)
        for k in range(HID):
            acc = acc + w2r[k] * h1[k]
        hwb[i] = acc
        pubb[i] = acc * y
        return carry
    lax.fori_loop(0, NODES_PER_TILE, loc2, 0)
    pltpu.sync_copy(pubb, t_src.at[nd])
    pltpu.sync_copy(zb, t_acc.at[nd])
    plsc.subcore_barrier()
    scope_mid.__exit__(None, None, None)

    with jax.named_scope("l2_edges"):
        edge_pass()
    plsc.subcore_barrier()

    scope_fin = jax.named_scope("final")
    scope_fin.__enter__()
    pltpu.sync_copy(t_acc.at[nd], accb)

    lanes = lax.iota(jnp.int32, LANES)
    lo = lanes < HID

    def loc4(i, carry):
        y = dinvb[i]
        o = y * accb[i] + (y * y) * hwb[i] + b2
        plsc.store_scatter(outb, [i * HID + lanes], o, mask=lo)
        return carry
    lax.fori_loop(0, NODES_PER_TILE, loc4, 0)

    @pl.when(c == 0)
    def _():
        pltpu.sync_copy(outb,
                        out_hbm.at[pl.ds(s * NODES_PER_TILE * HID,
                                         NODES_PER_TILE * HID)])
    scope_fin.__exit__(None, None, None)


def _gcn_sc(edge_index, x16g, W1, b1, W2, b2):
    mesh = plsc.VectorSubcoreMesh(core_axis_name="c", subcore_axis_name="s",
                                  num_cores=1)
    f32 = jnp.float32
    kern = pl.kernel(
        _gcn_body,
        out_type=jax.ShapeDtypeStruct((N_NODES * HID,), f32),
        mesh=mesh,
        compiler_params=pltpu.CompilerParams(needs_layout_passes=False,
                                             use_tc_tiling_on_sc=False),
        scratch_types=[
            pltpu.VMEM((EDGES_PER_TILE,), jnp.int32),
            pltpu.VMEM((EDGES_PER_TILE,), jnp.int32),
            pltpu.VMEM((EDGES_PER_TILE, LANES), f32),
            pltpu.VMEM((NODES_PER_TILE, LANES), f32),
            pltpu.VMEM((NODES_PER_TILE * IN_DIM,), f32),
            pltpu.VMEM((14, LANES), f32),
            pltpu.VMEM((NODES_PER_TILE, LANES), f32),
            pltpu.VMEM((NODES_PER_TILE, LANES), f32),
            pltpu.VMEM((NODES_PER_TILE, LANES), f32),
            pltpu.VMEM((NODES_PER_TILE, LANES), f32),
            pltpu.VMEM((NODES_PER_TILE, LANES), f32),
            pltpu.VMEM((NODES_PER_TILE, LANES), f32),
            pltpu.VMEM((NODES_PER_TILE, LANES), f32),
            pltpu.VMEM((NODES_PER_TILE * HID,), f32),
            pltpu.VMEM_SHARED((N_NODES, LANES), f32),
            pltpu.VMEM_SHARED((N_NODES, LANES), f32),
            pltpu.SemaphoreType.DMA,
            pltpu.SemaphoreType.DMA,
        ],
    )
    return kern(edge_index, x16g, W1, b1, W2, b2)


KBLK = 512
NSLOT = 6


def _mlp_body(v_ref, w1_hbm, b1_ref, w3_ref, b3_ref, o_ref, wbuf, sems):
    n_in = w1_hbm.shape[0]
    n_hidden = w1_hbm.shape[1]
    n_slab = n_in // KBLK

    def fire(i):
        return pltpu.async_copy(
            w1_hbm.at[pl.ds(i * KBLK, KBLK), :], wbuf.at[i % NSLOT],
            sems.at[i % NSLOT])

    descs = [fire(i) for i in range(NSLOT)]
    u = jnp.zeros((1, n_hidden), jnp.float32)
    for i in range(n_slab):
        descs[i % NSLOT].wait()
        vblk = v_ref[:, pl.ds(i * KBLK, KBLK)]
        u = u + jnp.dot(vblk, wbuf[i % NSLOT],
                        preferred_element_type=jnp.float32)
        if i + NSLOT < n_slab:
            descs[i % NSLOT] = fire(i + NSLOT)
    u = jnp.maximum(u + b1_ref[...], 0.0)
    z = jnp.dot(u, w3_ref[...], preferred_element_type=jnp.float32)
    z = z + b3_ref[...]
    z = z - jnp.max(z, axis=-1, keepdims=True)
    e = jnp.exp(z)
    o_ref[...] = e / jnp.sum(e, axis=-1, keepdims=True)


def _mlp_tc(v, lin1_W, lin1_b, lin3_W, lin3_b):
    n_hidden = lin1_W.shape[1]
    n_out = lin3_W.shape[1]
    return pl.pallas_call(
        _mlp_body,
        in_specs=[
            pl.BlockSpec(memory_space=pltpu.VMEM),
            pl.BlockSpec(memory_space=pl.ANY),
            pl.BlockSpec(memory_space=pltpu.VMEM),
            pl.BlockSpec(memory_space=pltpu.VMEM),
            pl.BlockSpec(memory_space=pltpu.VMEM),
        ],
        out_specs=pl.BlockSpec(memory_space=pltpu.VMEM),
        out_shape=jax.ShapeDtypeStruct((1, n_out), jnp.float32),
        compiler_params=pltpu.CompilerParams(
            vmem_limit_bytes=64 * 1024 * 1024),
        scratch_shapes=[
            pltpu.VMEM((NSLOT, KBLK, n_hidden), jnp.float32),
            pltpu.SemaphoreType.DMA((NSLOT,)),
        ],
    )(v, lin1_W, lin1_b, lin3_W, lin3_b)


def kernel(x, edge_index, W1, b1, W2, b2, lin1_W, lin1_b, lin3_W, lin3_b):
    x16g = x.reshape(N_SUB, NODES_PER_TILE * IN_DIM)
    h2 = _gcn_sc(edge_index, x16g, W1, b1, W2, b2)
    v = h2.reshape(1, -1)
    out = _mlp_tc(v, lin1_W, lin1_b.reshape(1, -1), lin3_W, lin3_b.reshape(1, -1))
    return out.reshape(-1)
